# baseline (speedup 1.0000x reference)
import functools

import jax
import jax.numpy as jnp
from jax import lax
from jax.experimental import pallas as pl
from jax.experimental.pallas import tpu as pltpu
from jax.experimental.pallas import tpu_sc as plsc

_BATCH, _HIST, _EMB = 4096, 50, 128
_N = _BATCH * _HIST
_NC, _NS = 2, 16
_NW = _NC * _NS
_RPW = _N // _NW
_BROWS = _BATCH // _NW
_GROW = 8
_GIDX = _GROW * _HIST
_NGRP = _BROWS // _GROW
_HALF = _NGRP // 2


def _gather_body(idx_hbm, table_hbm, out_hbm, idx_v, buf0, buf1,
                 gsem0, gsem1, osem0, osem1):
    wid = lax.axis_index("s") * _NC + lax.axis_index("c")
    row0 = wid * _BROWS
    pltpu.sync_copy(idx_hbm.at[wid], idx_v)

    def start_gather(g, buf, sem):
        pltpu.async_copy(table_hbm.at[idx_v.at[pl.ds(g * _GIDX, _GIDX)]],
                         buf, sem)

    def wait_gather(buf, sem):
        pltpu.make_async_copy(table_hbm.at[idx_v.at[pl.ds(0, _GIDX)]],
                              buf, sem).wait()

    def start_out(g, buf, sem):
        for b in range(_GROW):
            pltpu.async_copy(buf.at[pl.ds(b * _HIST, _HIST)],
                             out_hbm.at[row0 + g * _GROW + b], sem)

    def wait_out(buf, sem):
        for b in range(_GROW):
            pltpu.make_async_copy(buf.at[pl.ds(b * _HIST, _HIST)],
                                  out_hbm.at[row0 + b], sem).wait()

    start_gather(0, buf0, gsem0)

    def pair(h, carry):
        g0 = 2 * h
        wait_gather(buf0, gsem0)
        start_gather(g0 + 1, buf1, gsem1)
        start_out(g0, buf0, osem0)
        wait_gather(buf1, gsem1)
        wait_out(buf0, osem0)
        start_gather(lax.rem(g0 + 2, _NGRP), buf0, gsem0)
        start_out(g0 + 1, buf1, osem1)
        wait_out(buf1, osem1)
        return carry

    lax.fori_loop(0, _HALF, pair, 0)
    wait_gather(buf0, gsem0)


def kernel(x, table):
    idx = x.reshape(_NW, _RPW).astype(jnp.int32)
    mesh = plsc.VectorSubcoreMesh(core_axis_name="c", subcore_axis_name="s")
    run = functools.partial(
        pl.kernel,
        mesh=mesh,
        out_type=jax.ShapeDtypeStruct((_BATCH, _HIST, _EMB), jnp.float32),
        scratch_types=[
            pltpu.VMEM((_RPW,), jnp.int32),
            pltpu.VMEM((_GIDX, _EMB), jnp.float32),
            pltpu.VMEM((_GIDX, _EMB), jnp.float32),
            pltpu.SemaphoreType.DMA,
            pltpu.SemaphoreType.DMA,
            pltpu.SemaphoreType.DMA,
            pltpu.SemaphoreType.DMA,
        ],
    )(_gather_body)
    return run(idx, table)


# speedup vs baseline: 1.0042x; 1.0042x over previous
import functools

import jax
import jax.numpy as jnp
from jax import lax
from jax.experimental import pallas as pl
from jax.experimental.pallas import tpu as pltpu
from jax.experimental.pallas import tpu_sc as plsc

_BATCH, _HIST, _EMB = 4096, 50, 128
_HPAD = 56
_NC, _NS = 2, 16
_NW = _NC * _NS
_BROWS = _BATCH // _NW
_RPW = _BROWS * _HPAD
_GROW = 8
_GIDX = _GROW * _HIST
_NGRP = _BROWS // _GROW
_HALF = _NGRP // 2


def _gather_body(idx_hbm, table_hbm, out_hbm, idx_v, buf0, buf1,
                 gsem0, gsem1, osem0, osem1):
    wid = lax.axis_index("s") * _NC + lax.axis_index("c")
    row0 = wid * _BROWS
    pltpu.sync_copy(idx_hbm.at[wid], idx_v)

    def start_gather(g, buf, sem):
        for b in range(_GROW):
            pltpu.async_copy(
                table_hbm.at[idx_v.at[pl.ds((g * _GROW + b) * _HPAD, _HIST)]],
                buf.at[b], sem)

    def wait_gather(buf, sem):
        for b in range(_GROW):
            pltpu.make_async_copy(
                table_hbm.at[idx_v.at[pl.ds(b * _HPAD, _HIST)]],
                buf.at[b], sem).wait()

    def start_out(g, buf, sem):
        pltpu.async_copy(buf, out_hbm.at[pl.ds(row0 + g * _GROW, _GROW)], sem)

    def wait_out(buf, sem):
        pltpu.make_async_copy(buf, out_hbm.at[pl.ds(row0, _GROW)], sem).wait()

    start_gather(0, buf0, gsem0)

    def pair(h, carry):
        g0 = 2 * h
        wait_gather(buf0, gsem0)
        start_gather(g0 + 1, buf1, gsem1)
        start_out(g0, buf0, osem0)
        wait_gather(buf1, gsem1)
        wait_out(buf0, osem0)
        start_gather(lax.rem(g0 + 2, _NGRP), buf0, gsem0)
        start_out(g0 + 1, buf1, osem1)
        wait_out(buf1, osem1)
        return carry

    lax.fori_loop(0, _HALF, pair, 0)
    wait_gather(buf0, gsem0)


def kernel(x, table):
    idx = jnp.pad(x.astype(jnp.int32), ((0, 0), (0, _HPAD - _HIST)))
    idx = idx.reshape(_NW, _RPW)
    mesh = plsc.VectorSubcoreMesh(core_axis_name="c", subcore_axis_name="s")
    run = functools.partial(
        pl.kernel,
        mesh=mesh,
        out_type=jax.ShapeDtypeStruct((_BATCH, _HIST, _EMB), jnp.float32),
        scratch_types=[
            pltpu.VMEM((_RPW,), jnp.int32),
            pltpu.VMEM((_GROW, _HIST, _EMB), jnp.float32),
            pltpu.VMEM((_GROW, _HIST, _EMB), jnp.float32),
            pltpu.SemaphoreType.DMA,
            pltpu.SemaphoreType.DMA,
            pltpu.SemaphoreType.DMA,
            pltpu.SemaphoreType.DMA,
        ],
    )(_gather_body)
    return run(idx, table)
